# baseline (speedup 1.0000x reference)
import functools

import jax
import jax.numpy as jnp
from jax import lax
from jax.experimental import pallas as pl
from jax.experimental.pallas import tpu as pltpu
from jax.experimental.pallas import tpu_sc as plsc

B = 16384
D = 64
PATHS = 100
NC = 2
NS = 16
L = 16
NW = NC * NS
BPW = B // NW
CH = 128
NCH = BPW // CH
NG = BPW // L

_mesh = plsc.VectorSubcoreMesh(core_axis_name="c", subcore_axis_name="s")


@functools.partial(
    pl.kernel,
    mesh=_mesh,
    out_type=jax.ShapeDtypeStruct((B,), jnp.float32),
    scratch_types=[
        pltpu.VMEM((BPW,), jnp.int32),
        pltpu.VMEM((BPW,), jnp.int32),
        pltpu.VMEM((BPW,), jnp.int32),
        pltpu.VMEM((BPW, D), jnp.float32),
        pltpu.VMEM((BPW, D), jnp.float32),
        pltpu.VMEM((PATHS, D), jnp.float32),
        pltpu.VMEM((BPW,), jnp.float32),
        pltpu.SemaphoreType.DMA,
    ],
    compiler_params=pltpu.CompilerParams(
        needs_layout_passes=False, use_tc_tiling_on_sc=False),
)
def _hin2vec_sc(sn_hbm, en_hbm, pt_hbm, ntab_hbm, ptab_hbm, out_hbm,
                sidx, eidx, pidx, srows, erows, ptab, outv, sem):
    wid = lax.axis_index("s") * NC + lax.axis_index("c")
    base = wid * BPW

    pltpu.sync_copy(sn_hbm.at[pl.ds(base, BPW)], sidx)
    pltpu.sync_copy(en_hbm.at[pl.ds(base, BPW)], eidx)
    pltpu.sync_copy(pt_hbm.at[pl.ds(base, BPW)], pidx)
    pltpu.sync_copy(ptab_hbm, ptab)

    copies = []
    for j in range(NCH):
        sl = pl.ds(j * CH, CH)
        copies.append(pltpu.async_copy(ntab_hbm.at[sidx.at[sl]], srows.at[sl], sem))
        copies.append(pltpu.async_copy(ntab_hbm.at[eidx.at[sl]], erows.at[sl], sem))
    for c in copies:
        c.wait()

    lanes = lax.iota(jnp.int32, L)

    def group_body(g, carry):
        rows = g * L + lanes
        pvec = pidx[pl.ds(g * L, L)]
        acc = jnp.zeros((L,), jnp.float32)
        for d in range(D):
            col = jnp.full((L,), d, jnp.int32)
            sv = plsc.load_gather(srows, [rows, col])
            ev = plsc.load_gather(erows, [rows, col])
            pv = plsc.load_gather(ptab, [pvec, col])
            acc = acc + jnp.where(pv > 0.0, sv * ev, 0.0)
        z = jnp.exp(-jnp.abs(acc))
        sig = jnp.where(acc >= 0.0, 1.0 / (1.0 + z), z / (1.0 + z))
        outv[pl.ds(g * L, L)] = sig
        return carry

    lax.fori_loop(0, NG, group_body, 0)

    pltpu.sync_copy(outv, out_hbm.at[pl.ds(base, BPW)])


def kernel(start_node, end_node, path, node_table, path_table):
    return _hin2vec_sc(
        start_node.astype(jnp.int32),
        end_node.astype(jnp.int32),
        path.astype(jnp.int32),
        node_table,
        path_table,
    )
